# Initial kernel scaffold; baseline (speedup 1.0000x reference)
import jax
import jax.numpy as jnp
from jax.experimental import pallas as pl


def kernel(x, zeros):
    raise NotImplementedError("write your pallas kernel here")


# speedup vs baseline: 192.7438x; 192.7438x over previous
import functools

import jax
import jax.numpy as jnp
from jax import lax
from jax.experimental import pallas as pl
from jax.experimental.pallas import tpu as pltpu
from jax.experimental.pallas import tpu_sc as plsc

_NC = 2
_NS = 16
_NW = _NC * _NS
_L = 16


def _softmax_body(z_ref, p_ref):
    z = z_ref[...]
    m = jnp.max(z)
    e = jnp.exp(z - m)
    p_ref[...] = e / jnp.sum(e)


@functools.lru_cache(maxsize=None)
def _make_gather(n_table: int, total: int):
    assert total % _NW == 0
    per_w = total // _NW
    ch = per_w
    n_ch = 1
    budget = (524284 - 4 * n_table - 4096) // 8
    while ch > budget or ch % _L:
        n_ch *= 2
        assert per_w % n_ch == 0
        ch = per_w // n_ch

    mesh = plsc.VectorSubcoreMesh(core_axis_name="c", subcore_axis_name="s")

    @functools.partial(
        pl.kernel,
        out_type=jax.ShapeDtypeStruct((total,), jnp.float32),
        mesh=mesh,
        scratch_types=[
            pltpu.VMEM((n_table,), jnp.float32),
            pltpu.VMEM((ch,), jnp.int32),
            pltpu.VMEM((ch,), jnp.float32),
        ],
        compiler_params=pltpu.CompilerParams(needs_layout_passes=False),
    )
    def gather_k(table_hbm, xf_hbm, out_hbm, table_v, idx_v, out_v):
        wid = lax.axis_index("s") * _NC + lax.axis_index("c")
        base = wid * per_w
        pltpu.sync_copy(table_hbm, table_v)
        for c in range(n_ch):
            off = base + c * ch
            pltpu.sync_copy(xf_hbm.at[pl.ds(off, ch)], idx_v)

            def body(i, _):
                idx = idx_v[pl.ds(i * _L, _L)]
                out_v[pl.ds(i * _L, _L)] = plsc.load_gather(table_v, [idx])
                return 0

            lax.fori_loop(0, ch // _L, body, 0)
            pltpu.sync_copy(out_v, out_hbm.at[pl.ds(off, ch)])

    return gather_k


def kernel(x, zeros):
    n = zeros.shape[0]
    rows = 8
    probs = pl.pallas_call(
        _softmax_body,
        out_shape=jax.ShapeDtypeStruct((rows, n // rows), jnp.float32),
    )(zeros.reshape(rows, n // rows)).reshape(n)
    xf = x.reshape(-1)
    out = _make_gather(n, xf.size)(probs, xf)
    return out.reshape(x.shape)


# speedup vs baseline: 246.9129x; 1.2810x over previous
import functools

import jax
import jax.numpy as jnp
from jax import lax
from jax.experimental import pallas as pl
from jax.experimental.pallas import tpu as pltpu
from jax.experimental.pallas import tpu_sc as plsc

_NC = 2
_NS = 16
_NW = _NC * _NS
_L = 16


def _softmax_body(z_ref, p_ref):
    z = z_ref[...]
    m = jnp.max(z)
    e = jnp.exp(z - m)
    p_ref[...] = e / jnp.sum(e)


@functools.lru_cache(maxsize=None)
def _make_gather(n_table: int, total: int):
    assert total % _NW == 0
    per_w = total // _NW
    ch = per_w
    n_ch = 1
    budget = (524284 - 4 * n_table - 4096) // 16
    while ch > budget or ch % (8 * _L):
        n_ch *= 2
        assert per_w % n_ch == 0
        ch = per_w // n_ch
    n_ch = max(n_ch, 2)
    ch = per_w // n_ch

    mesh = plsc.VectorSubcoreMesh(core_axis_name="c", subcore_axis_name="s")

    @functools.partial(
        pl.kernel,
        out_type=jax.ShapeDtypeStruct((total,), jnp.float32),
        mesh=mesh,
        scratch_types=[
            pltpu.VMEM((n_table,), jnp.float32),
            pltpu.VMEM((2, ch), jnp.int32),
            pltpu.VMEM((2, ch), jnp.float32),
            pltpu.SemaphoreType.DMA,
            pltpu.SemaphoreType.DMA((2,)),
            pltpu.SemaphoreType.DMA((2,)),
        ],
        compiler_params=pltpu.CompilerParams(needs_layout_passes=False),
    )
    def gather_k(table_hbm, xf_hbm, out_hbm, table_v, idx_v, out_v,
                 sem_tab, sem_in, sem_out):
        wid = lax.axis_index("s") * _NC + lax.axis_index("c")
        base = wid * per_w
        tab = pltpu.async_copy(table_hbm, table_v, sem_tab)
        d_in, d_out = {}, {}

        def start_in(c):
            b = c % 2
            d_in[c] = pltpu.async_copy(
                xf_hbm.at[pl.ds(base + c * ch, ch)], idx_v.at[b],
                sem_in.at[b])

        start_in(0)
        start_in(1)
        tab.wait()
        for c in range(n_ch):
            b = c % 2
            d_in[c].wait()
            if c >= 2:
                d_out[c - 2].wait()

            @plsc.parallel_loop(0, ch, _L, unroll=8)
            def _(i, b=b):
                idx = idx_v[b, pl.ds(i, _L)]
                out_v[b, pl.ds(i, _L)] = plsc.load_gather(table_v, [idx])

            d_out[c] = pltpu.async_copy(
                out_v.at[b], out_hbm.at[pl.ds(base + c * ch, ch)],
                sem_out.at[b])
            if c + 2 < n_ch:
                start_in(c + 2)
        d_out[n_ch - 2].wait()
        d_out[n_ch - 1].wait()

    return gather_k


def kernel(x, zeros):
    n = zeros.shape[0]
    rows = 8
    probs = pl.pallas_call(
        _softmax_body,
        out_shape=jax.ShapeDtypeStruct((rows, n // rows), jnp.float32),
    )(zeros.reshape(rows, n // rows)).reshape(n)
    xf = x.reshape(-1)
    out = _make_gather(n, xf.size)(probs, xf)
    return out.reshape(x.shape)


# speedup vs baseline: 379.3932x; 1.5365x over previous
import functools

import jax
import jax.numpy as jnp
from jax import lax
from jax.experimental import pallas as pl
from jax.experimental.pallas import tpu as pltpu
from jax.experimental.pallas import tpu_sc as plsc

_NC = 2
_NS = 16
_NW = _NC * _NS
_L = 16


def _softmax_body(z_ref, p_ref):
    z = z_ref[...]
    m = jnp.max(z)
    e = jnp.exp(z - m)
    p_ref[...] = e / jnp.sum(e)


@functools.lru_cache(maxsize=None)
def _make_gather(n_table: int, rows: int, cols: int):
    assert rows % _NW == 0
    rows_w = rows // _NW
    ch = rows_w
    n_ch = 1
    budget = (524284 - 4 * n_table - 32768) // (16 * cols)
    while ch > budget:
        n_ch *= 2
        assert rows_w % n_ch == 0
        ch = rows_w // n_ch
    n_ch = max(n_ch, 2)
    ch = rows_w // n_ch
    offs = list(range(0, cols - _L + 1, _L))
    if cols % _L:
        offs.append(cols - _L)
    assert n_ch >= 4 and n_ch % 2 == 0

    mesh = plsc.VectorSubcoreMesh(core_axis_name="c", subcore_axis_name="s")

    @functools.partial(
        pl.kernel,
        out_type=jax.ShapeDtypeStruct((rows, cols), jnp.float32),
        mesh=mesh,
        scratch_types=[
            pltpu.VMEM((n_table,), jnp.float32),
            pltpu.VMEM((2, ch, cols), jnp.int32),
            pltpu.VMEM((2, ch, cols), jnp.float32),
            pltpu.SemaphoreType.DMA,
            pltpu.SemaphoreType.DMA((2,)),
            pltpu.SemaphoreType.DMA((2,)),
        ],
        compiler_params=pltpu.CompilerParams(needs_layout_passes=False),
    )
    def gather_k(table_hbm, x_hbm, out_hbm, table_v, idx_v, out_v,
                 sem_tab, sem_in, sem_out):
        wid = lax.axis_index("s") * _NC + lax.axis_index("c")
        base = wid * rows_w
        tab = pltpu.async_copy(table_hbm, table_v, sem_tab)

        def in_cp(c, b):
            return pltpu.make_async_copy(
                x_hbm.at[pl.ds(base + c * ch, ch), :], idx_v.at[b],
                sem_in.at[b])

        def out_cp(c, b):
            return pltpu.make_async_copy(
                out_v.at[b], out_hbm.at[pl.ds(base + c * ch, ch), :],
                sem_out.at[b])

        def do_chunk(c, b, first):
            in_cp(c, b).wait()
            if not first:
                out_cp(c - 2, b).wait()

            @plsc.parallel_loop(0, ch, 1, unroll=2)
            def _(r):
                for off in offs:
                    idx = idx_v[b, r, pl.ds(off, _L)]
                    out_v[b, r, pl.ds(off, _L)] = plsc.load_gather(
                        table_v, [idx])

            out_cp(c, b).start()

            @pl.when(c + 2 < n_ch)
            def _():
                in_cp(c + 2, b).start()

        in_cp(0, 0).start()
        in_cp(1, 1).start()
        tab.wait()
        do_chunk(0, 0, True)
        do_chunk(1, 1, True)

        def pair_body(p, carry):
            do_chunk(2 * p, 0, False)
            do_chunk(2 * p + 1, 1, False)
            return carry

        lax.fori_loop(1, n_ch // 2, pair_body, 0)
        out_cp(n_ch - 2, 0).wait()
        out_cp(n_ch - 1, 1).wait()

    return gather_k


def kernel(x, zeros):
    n = zeros.shape[0]
    probs = pl.pallas_call(
        _softmax_body,
        out_shape=jax.ShapeDtypeStruct((n,), jnp.float32),
    )(zeros)
    return _make_gather(n, x.shape[0], x.shape[1])(probs, x)
